# baseline (speedup 1.0000x reference)
import functools

import jax
import jax.numpy as jnp
import numpy as np
from jax.experimental import pallas as pl
from jax.experimental.pallas import tpu as pltpu
from jax.experimental.pallas import tpu_sc as plsc

H = 16
HD = 64
WINDOW = 8
NEG = -1e30
TR = 128
LW = 256
GMAX = 128

_INTERPRET = False


def _qkv_kernel(x_ref, w_ref, b_ref, gidx_ref, o_ref, og_ref):
    acc = (
        jnp.dot(x_ref[...], w_ref[...], preferred_element_type=jnp.float32)
        + b_ref[...]
    )
    out = acc.astype(o_ref.dtype)
    o_ref[...] = out
    m = x_ref.shape[0]
    g = gidx_ref.shape[1]
    gcol = jnp.transpose(gidx_ref[...])
    p = (jax.lax.broadcasted_iota(jnp.int32, (g, m), 1) == gcol)
    og_ref[...] = jnp.dot(
        p.astype(jnp.bfloat16), out, preferred_element_type=jnp.float32
    ).astype(og_ref.dtype)


def _qkv_matmul(x, w, b, gidx2, bn=1024):
    m, k = x.shape
    k2, n = w.shape
    g = gidx2.shape[1]
    grid = (n // bn,)
    return pl.pallas_call(
        _qkv_kernel,
        grid=grid,
        in_specs=[
            pl.BlockSpec((m, k), lambda j: (0, 0)),
            pl.BlockSpec((k, bn), lambda j: (0, j)),
            pl.BlockSpec((1, bn), lambda j: (0, j)),
            pl.BlockSpec((1, g), lambda j: (0, 0)),
        ],
        out_specs=[
            pl.BlockSpec((m, bn), lambda j: (0, j)),
            pl.BlockSpec((g, bn), lambda j: (0, j)),
        ],
        out_shape=[
            jax.ShapeDtypeStruct((m, n), jnp.bfloat16),
            jax.ShapeDtypeStruct((g, n), jnp.bfloat16),
        ],
        interpret=_INTERPRET,
    )(x, w, b, gidx2)


def _matmul_kernel(x_ref, w_ref, b_ref, o_ref):
    acc = (
        jnp.dot(x_ref[...], w_ref[...], preferred_element_type=jnp.float32)
        + b_ref[...]
    )
    o_ref[...] = acc.astype(o_ref.dtype)


def _matmul(x, w, b, out_dtype=jnp.float32, bn=1024):
    m, k = x.shape
    k2, n = w.shape
    grid = (n // bn,)
    return pl.pallas_call(
        _matmul_kernel,
        grid=grid,
        in_specs=[
            pl.BlockSpec((m, k), lambda j: (0, 0)),
            pl.BlockSpec((k, bn), lambda j: (0, j)),
            pl.BlockSpec((1, bn), lambda j: (0, j)),
        ],
        out_specs=pl.BlockSpec((m, bn), lambda j: (0, j)),
        out_shape=jax.ShapeDtypeStruct((m, n), out_dtype),
        interpret=_INTERPRET,
    )(x, w, b)


def _band_consts(t):
    out = np.full((3, TR, LW), NEG, np.float32)
    for s, off in enumerate((0, (LW - TR) // 2, LW - TR)):
        i = np.arange(TR)[:, None]
        j = np.arange(LW)[None, :]
        out[s] = np.where(np.abs(j - i - off) <= WINDOW, 0.0, NEG)
    return jnp.asarray(out)


def _attn_kernel(q_ref, k_ref, v_ref, kg_ref, vg_ref, gvneg_ref, bandc_ref,
                 o_ref, *, t):
    h = pl.program_id(0)
    nr = t // TR

    q = q_ref[...].reshape(t, HD) * jnp.bfloat16(1.0 / np.sqrt(HD))
    k = k_ref[...].reshape(t, HD)
    v = v_ref[...].reshape(t, HD)
    kg = kg_ref[...].reshape(kg_ref.shape[0], HD)
    vg = vg_ref[...].reshape(vg_ref.shape[0], HD)
    gvneg = gvneg_ref[...]

    for r in range(nr):
        t0 = r * TR
        ls = min(max(t0 - (LW - TR) // 2, 0), t - LW)
        sel = 0 if r == 0 else (2 if r == nr - 1 else 1)
        qs = q[t0:t0 + TR]
        kl = k[ls:ls + LW]
        vl = v[ls:ls + LW]

        s_loc = jax.lax.dot_general(
            qs, kl, (((1,), (1,)), ((), ())),
            preferred_element_type=jnp.float32,
        ) + bandc_ref[sel]

        s_g = jax.lax.dot_general(
            qs, kg, (((1,), (1,)), ((), ())),
            preferred_element_type=jnp.float32,
        ) + gvneg

        m = jnp.maximum(
            jnp.max(s_loc, axis=1, keepdims=True),
            jnp.max(s_g, axis=1, keepdims=True),
        )
        p_loc = jnp.exp(s_loc - m)
        p_g = jnp.exp(s_g - m)
        l = (jnp.sum(p_loc, axis=1, keepdims=True)
             + jnp.sum(p_g, axis=1, keepdims=True))
        acc = (
            jnp.dot(p_loc.astype(jnp.bfloat16), vl,
                    preferred_element_type=jnp.float32)
            + jnp.dot(p_g.astype(jnp.bfloat16), vg,
                      preferred_element_type=jnp.float32)
        )
        o_ref[pl.ds(t0, TR), 0, 0, :] = (acc / l).astype(o_ref.dtype)


def _attention(qkv, kvsrc, gvalid, bandc):
    t = qkv.shape[0]
    ng = gvalid.shape[1]
    gvneg = jnp.where(gvalid > 0.0, 0.0, NEG).astype(jnp.float32)
    qkv4 = qkv.reshape(t, 3 * H, 1, HD)
    kvsrc4 = kvsrc.reshape(kvsrc.shape[0], 3 * H, 1, HD)
    grid = (H,)
    ctx4 = pl.pallas_call(
        functools.partial(_attn_kernel, t=t),
        grid=grid,
        in_specs=[
            pl.BlockSpec((t, 1, 1, HD), lambda h: (0, h, 0, 0)),
            pl.BlockSpec((t, 1, 1, HD), lambda h: (0, H + h, 0, 0)),
            pl.BlockSpec((t, 1, 1, HD), lambda h: (0, 2 * H + h, 0, 0)),
            pl.BlockSpec((ng, 1, 1, HD), lambda h: (0, H + h, 0, 0)),
            pl.BlockSpec((ng, 1, 1, HD), lambda h: (0, 2 * H + h, 0, 0)),
            pl.BlockSpec((1, ng), lambda h: (0, 0)),
            pl.BlockSpec((3, TR, LW), lambda h: (0, 0, 0)),
        ],
        out_specs=pl.BlockSpec((t, 1, 1, HD), lambda h: (0, h, 0, 0)),
        out_shape=jax.ShapeDtypeStruct((t, H, 1, HD), jnp.bfloat16),
        interpret=_INTERPRET,
    )(qkv4, qkv4, qkv4, kvsrc4, kvsrc4, gvneg, bandc)
    return ctx4.reshape(t, H * HD)


def kernel(x, global_mask, Wq, bq, Wk, bk, Wv, bv, Wo, bo):
    b, t, d = x.shape
    x2 = x[0].astype(jnp.bfloat16)
    wqkv = jnp.concatenate([Wq, Wk, Wv], axis=1).astype(jnp.bfloat16)
    bqkv = jnp.concatenate([bq, bk, bv])[None, :]

    mask = global_mask[0]
    csum = jnp.cumsum(mask.astype(jnp.int32))
    g = csum[-1]
    slots = jnp.where(mask, csum - 1, GMAX + t)
    gidx = (
        jnp.zeros((GMAX,), jnp.int32)
        .at[slots]
        .set(jnp.arange(t, dtype=jnp.int32), mode="drop")
    )
    gvalid_fast = (jnp.arange(GMAX) < g).astype(jnp.float32)[None, :]
    gvalid_slow = mask.astype(jnp.float32)[None, :]

    qkv, kv_glob = _qkv_matmul(x2, wqkv, bqkv, gidx[None, :])

    bandc = _band_consts(t)

    def fast(qkv_):
        return _attention(qkv_, kv_glob, gvalid_fast, bandc)

    def slow(qkv_):
        return _attention(qkv_, qkv_, gvalid_slow, bandc)

    ctx = jax.lax.cond(g <= GMAX, fast, slow, qkv)
    out2 = _matmul(ctx, Wo.astype(jnp.bfloat16), bo[None, :])
    return out2[None]


# speedup vs baseline: 1.1070x; 1.1070x over previous
import functools

import jax
import jax.numpy as jnp
import numpy as np
from jax.experimental import pallas as pl
from jax.experimental.pallas import tpu as pltpu
from jax.experimental.pallas import tpu_sc as plsc

H = 16
HD = 64
WINDOW = 8
NEG = -1e30
TR = 256
LW = 384
GMAX = 128

_INTERPRET = False


def _qkv_kernel(x_ref, w_ref, b_ref, gidx_ref, o_ref, og_ref):
    acc = (
        jnp.dot(x_ref[...], w_ref[...], preferred_element_type=jnp.float32)
        + b_ref[...]
    )
    out = acc.astype(o_ref.dtype)
    o_ref[...] = out
    m = x_ref.shape[0]
    g = gidx_ref.shape[1]
    gcol = jnp.transpose(gidx_ref[...])
    p = (jax.lax.broadcasted_iota(jnp.int32, (g, m), 1) == gcol)
    og_ref[...] = jnp.dot(
        p.astype(jnp.bfloat16), out, preferred_element_type=jnp.float32
    ).astype(og_ref.dtype)


def _qkv_matmul(x, w, b, gidx2, bn=1024):
    m, k = x.shape
    k2, n = w.shape
    g = gidx2.shape[1]
    grid = (n // bn,)
    return pl.pallas_call(
        _qkv_kernel,
        grid=grid,
        in_specs=[
            pl.BlockSpec((m, k), lambda j: (0, 0)),
            pl.BlockSpec((k, bn), lambda j: (0, j)),
            pl.BlockSpec((1, bn), lambda j: (0, j)),
            pl.BlockSpec((1, g), lambda j: (0, 0)),
        ],
        out_specs=[
            pl.BlockSpec((m, bn), lambda j: (0, j)),
            pl.BlockSpec((g, bn), lambda j: (0, j)),
        ],
        out_shape=[
            jax.ShapeDtypeStruct((m, n), jnp.bfloat16),
            jax.ShapeDtypeStruct((g, n), jnp.bfloat16),
        ],
        interpret=_INTERPRET,
    )(x, w, b, gidx2)


def _matmul_kernel(x_ref, w_ref, b_ref, o_ref):
    acc = (
        jnp.dot(x_ref[...], w_ref[...], preferred_element_type=jnp.float32)
        + b_ref[...]
    )
    o_ref[...] = acc.astype(o_ref.dtype)


def _matmul(x, w, b, out_dtype=jnp.float32, bn=1024):
    m, k = x.shape
    k2, n = w.shape
    grid = (n // bn,)
    return pl.pallas_call(
        _matmul_kernel,
        grid=grid,
        in_specs=[
            pl.BlockSpec((m, k), lambda j: (0, 0)),
            pl.BlockSpec((k, bn), lambda j: (0, j)),
            pl.BlockSpec((1, bn), lambda j: (0, j)),
        ],
        out_specs=pl.BlockSpec((m, bn), lambda j: (0, j)),
        out_shape=jax.ShapeDtypeStruct((m, n), out_dtype),
        interpret=_INTERPRET,
    )(x, w, b)


def _band_consts(t):
    out = np.full((3, TR, LW), NEG, np.float32)
    for s, off in enumerate((0, (LW - TR) // 2, LW - TR)):
        i = np.arange(TR)[:, None]
        j = np.arange(LW)[None, :]
        out[s] = np.where(np.abs(j - i - off) <= WINDOW, 0.0, NEG)
    return jnp.asarray(out)


def _attn_kernel(q_ref, k_ref, v_ref, kg_ref, vg_ref, gvneg_ref, bandc_ref,
                 wo_ref, bo_ref, o_ref, *, t):
    h = pl.program_id(0)
    nr = t // TR

    q = q_ref[...].reshape(t, HD) * jnp.bfloat16(1.0 / np.sqrt(HD))
    k = k_ref[...].reshape(t, HD)
    v = v_ref[...].reshape(t, HD)
    kg = kg_ref[...].reshape(kg_ref.shape[0], HD)
    vg = vg_ref[...].reshape(vg_ref.shape[0], HD)
    gvneg = gvneg_ref[...]

    ctx_rows = []
    for r in range(nr):
        t0 = r * TR
        ls = min(max(t0 - (LW - TR) // 2, 0), t - LW)
        sel = 0 if r == 0 else (2 if r == nr - 1 else 1)
        qs = q[t0:t0 + TR]
        kl = k[ls:ls + LW]
        vl = v[ls:ls + LW]

        s_loc = jax.lax.dot_general(
            qs, kl, (((1,), (1,)), ((), ())),
            preferred_element_type=jnp.float32,
        ) + bandc_ref[sel]

        s_g = jax.lax.dot_general(
            qs, kg, (((1,), (1,)), ((), ())),
            preferred_element_type=jnp.float32,
        ) + gvneg

        m = jnp.maximum(
            jnp.max(s_loc, axis=1, keepdims=True),
            jnp.max(s_g, axis=1, keepdims=True),
        )
        p_loc = jnp.exp(s_loc - m)
        p_g = jnp.exp(s_g - m)
        l = (jnp.sum(p_loc, axis=1, keepdims=True)
             + jnp.sum(p_g, axis=1, keepdims=True))
        acc = (
            jnp.dot(p_loc.astype(jnp.bfloat16), vl,
                    preferred_element_type=jnp.float32)
            + jnp.dot(p_g.astype(jnp.bfloat16), vg,
                      preferred_element_type=jnp.float32)
        )
        ctx_rows.append(acc / l)

    ctx = jnp.concatenate(ctx_rows, axis=0)
    contrib = jnp.dot(ctx.astype(jnp.bfloat16), wo_ref[...],
                      preferred_element_type=jnp.float32)

    @pl.when(h == 0)
    def _init():
        o_ref[...] = contrib + bo_ref[...]

    @pl.when(h != 0)
    def _accum():
        o_ref[...] += contrib


def _attention(qkv, kvsrc, gvalid, bandc, wo, bo):
    t = qkv.shape[0]
    d = H * HD
    ng = gvalid.shape[1]
    gvneg = jnp.where(gvalid > 0.0, 0.0, NEG).astype(jnp.float32)
    qkv4 = qkv.reshape(t, 3 * H, 1, HD)
    kvsrc4 = kvsrc.reshape(kvsrc.shape[0], 3 * H, 1, HD)
    grid = (H,)
    return pl.pallas_call(
        functools.partial(_attn_kernel, t=t),
        grid=grid,
        in_specs=[
            pl.BlockSpec((t, 1, 1, HD), lambda h: (0, h, 0, 0)),
            pl.BlockSpec((t, 1, 1, HD), lambda h: (0, H + h, 0, 0)),
            pl.BlockSpec((t, 1, 1, HD), lambda h: (0, 2 * H + h, 0, 0)),
            pl.BlockSpec((ng, 1, 1, HD), lambda h: (0, H + h, 0, 0)),
            pl.BlockSpec((ng, 1, 1, HD), lambda h: (0, 2 * H + h, 0, 0)),
            pl.BlockSpec((1, ng), lambda h: (0, 0)),
            pl.BlockSpec((3, TR, LW), lambda h: (0, 0, 0)),
            pl.BlockSpec((HD, d), lambda h: (h, 0)),
            pl.BlockSpec((1, d), lambda h: (0, 0)),
        ],
        out_specs=pl.BlockSpec((t, d), lambda h: (0, 0)),
        out_shape=jax.ShapeDtypeStruct((t, d), jnp.float32),
        interpret=_INTERPRET,
    )(qkv4, qkv4, qkv4, kvsrc4, kvsrc4, gvneg, bandc, wo, bo)


def kernel(x, global_mask, Wq, bq, Wk, bk, Wv, bv, Wo, bo):
    b, t, d = x.shape
    x2 = x[0].astype(jnp.bfloat16)
    wqkv = jnp.concatenate([Wq, Wk, Wv], axis=1).astype(jnp.bfloat16)
    bqkv = jnp.concatenate([bq, bk, bv])[None, :]

    mask = global_mask[0]
    csum = jnp.cumsum(mask.astype(jnp.int32))
    g = csum[-1]
    slots = jnp.where(mask, csum - 1, GMAX + t)
    gidx = (
        jnp.zeros((GMAX,), jnp.int32)
        .at[slots]
        .set(jnp.arange(t, dtype=jnp.int32), mode="drop")
    )
    gvalid_fast = (jnp.arange(GMAX) < g).astype(jnp.float32)[None, :]
    gvalid_slow = mask.astype(jnp.float32)[None, :]

    qkv, kv_glob = _qkv_matmul(x2, wqkv, bqkv, gidx[None, :])

    bandc = _band_consts(t)
    wo_b = Wo.astype(jnp.bfloat16)
    bo_b = bo[None, :]

    def fast(qkv_):
        return _attention(qkv_, kv_glob, gvalid_fast, bandc, wo_b, bo_b)

    def slow(qkv_):
        return _attention(qkv_, qkv_, gvalid_slow, bandc, wo_b, bo_b)

    out2 = jax.lax.cond(g <= GMAX, fast, slow, qkv)
    return out2[None]


# speedup vs baseline: 1.1312x; 1.0218x over previous
import functools

import jax
import jax.numpy as jnp
import numpy as np
from jax.experimental import pallas as pl
from jax.experimental.pallas import tpu as pltpu
from jax.experimental.pallas import tpu_sc as plsc

H = 16
HD = 64
WINDOW = 8
NEG = -1e30
TR = 128
LW = 256
GMAX = 128

_INTERPRET = False


def _qkv_kernel(x_ref, w_ref, b_ref, gidx_ref, o_ref, og_ref):
    acc = (
        jnp.dot(x_ref[...], w_ref[...], preferred_element_type=jnp.float32)
        + b_ref[...]
    )
    out = acc.astype(o_ref.dtype)
    o_ref[...] = out
    m = x_ref.shape[0]
    g = gidx_ref.shape[1]
    gcol = jnp.transpose(gidx_ref[...])
    p = (jax.lax.broadcasted_iota(jnp.int32, (g, m), 1) == gcol)
    og_ref[...] = jnp.dot(
        p.astype(jnp.bfloat16), out, preferred_element_type=jnp.float32
    ).astype(og_ref.dtype)


def _qkv_matmul(x, w, b, gidx2, bn=1024):
    m, k = x.shape
    k2, n = w.shape
    g = gidx2.shape[1]
    grid = (n // bn,)
    return pl.pallas_call(
        _qkv_kernel,
        grid=grid,
        in_specs=[
            pl.BlockSpec((m, k), lambda j: (0, 0)),
            pl.BlockSpec((k, bn), lambda j: (0, j)),
            pl.BlockSpec((1, bn), lambda j: (0, j)),
            pl.BlockSpec((1, g), lambda j: (0, 0)),
        ],
        out_specs=[
            pl.BlockSpec((m, bn), lambda j: (0, j)),
            pl.BlockSpec((g, bn), lambda j: (0, j)),
        ],
        out_shape=[
            jax.ShapeDtypeStruct((m, n), jnp.bfloat16),
            jax.ShapeDtypeStruct((g, n), jnp.bfloat16),
        ],
        interpret=_INTERPRET,
    )(x, w, b, gidx2)


def _matmul_kernel(x_ref, w_ref, b_ref, o_ref):
    acc = (
        jnp.dot(x_ref[...], w_ref[...], preferred_element_type=jnp.float32)
        + b_ref[...]
    )
    o_ref[...] = acc.astype(o_ref.dtype)


def _matmul(x, w, b, out_dtype=jnp.float32, bn=1024):
    m, k = x.shape
    k2, n = w.shape
    grid = (n // bn,)
    return pl.pallas_call(
        _matmul_kernel,
        grid=grid,
        in_specs=[
            pl.BlockSpec((m, k), lambda j: (0, 0)),
            pl.BlockSpec((k, bn), lambda j: (0, j)),
            pl.BlockSpec((1, bn), lambda j: (0, j)),
        ],
        out_specs=pl.BlockSpec((m, bn), lambda j: (0, j)),
        out_shape=jax.ShapeDtypeStruct((m, n), out_dtype),
        interpret=_INTERPRET,
    )(x, w, b)


def _band_consts(t):
    out = np.full((3, TR, LW), NEG, np.float32)
    for s, off in enumerate((0, (LW - TR) // 2, LW - TR)):
        i = np.arange(TR)[:, None]
        j = np.arange(LW)[None, :]
        out[s] = np.where(np.abs(j - i - off) <= WINDOW, 0.0, NEG)
    return jnp.asarray(out)


def _attn_kernel(q_ref, k_ref, v_ref, kg_ref, vg_ref, gvneg_ref,
                 wo_ref, bo_ref, o_ref, *, t):
    h = pl.program_id(0)
    nr = t // TR

    scale = 1.0 / np.sqrt(HD)
    q = q_ref[...].reshape(t, HD)
    k = k_ref[...].reshape(t, HD)
    v = v_ref[...].reshape(t, HD)
    kg = kg_ref[...].reshape(kg_ref.shape[0], HD)
    vg = vg_ref[...].reshape(vg_ref.shape[0], HD)
    gv = gvneg_ref[...] > -1.0

    ctx_rows = []
    for r in range(nr):
        t0 = r * TR
        ls = min(max(t0 - (LW - TR) // 2, 0), t - LW)
        qs = q[t0:t0 + TR]
        kl = k[ls:ls + LW]
        vl = v[ls:ls + LW]

        row_ids = t0 + jax.lax.broadcasted_iota(jnp.int32, (TR, LW), 0)
        key_ids = ls + jax.lax.broadcasted_iota(jnp.int32, (TR, LW), 1)
        band = jnp.abs(key_ids - row_ids) <= WINDOW

        s_loc = jax.lax.dot_general(
            qs, kl, (((1,), (1,)), ((), ())),
            preferred_element_type=jnp.float32,
        ) * scale
        s_loc = jnp.where(band, s_loc, NEG)

        s_g = jax.lax.dot_general(
            qs, kg, (((1,), (1,)), ((), ())),
            preferred_element_type=jnp.float32,
        ) * scale
        s_g = jnp.where(gv, s_g, NEG)

        m = jnp.maximum(
            jnp.max(s_loc, axis=1, keepdims=True),
            jnp.max(s_g, axis=1, keepdims=True),
        )
        p_loc = jnp.exp(s_loc - m)
        p_g = jnp.exp(s_g - m)
        l = (jnp.sum(p_loc, axis=1, keepdims=True)
             + jnp.sum(p_g, axis=1, keepdims=True))
        acc = (
            jnp.dot(p_loc.astype(jnp.bfloat16), vl,
                    preferred_element_type=jnp.float32)
            + jnp.dot(p_g.astype(jnp.bfloat16), vg,
                      preferred_element_type=jnp.float32)
        )
        ctx_rows.append(acc / l)

    ctx = jnp.concatenate(ctx_rows, axis=0)
    contrib = jnp.dot(ctx.astype(jnp.bfloat16), wo_ref[...],
                      preferred_element_type=jnp.float32)

    @pl.when(h == 0)
    def _init():
        o_ref[...] = contrib + bo_ref[...]

    @pl.when(h != 0)
    def _accum():
        o_ref[...] += contrib


def _attention(qkv, kvsrc, gvalid, wo, bo):
    t = qkv.shape[0]
    d = H * HD
    ng = gvalid.shape[1]
    gvneg = jnp.where(gvalid > 0.0, 0.0, NEG).astype(jnp.float32)
    qkv4 = qkv.reshape(t, 3 * H, 1, HD)
    kvsrc4 = kvsrc.reshape(kvsrc.shape[0], 3 * H, 1, HD)
    grid = (H,)
    return pl.pallas_call(
        functools.partial(_attn_kernel, t=t),
        grid=grid,
        in_specs=[
            pl.BlockSpec((t, 1, 1, HD), lambda h: (0, h, 0, 0)),
            pl.BlockSpec((t, 1, 1, HD), lambda h: (0, H + h, 0, 0)),
            pl.BlockSpec((t, 1, 1, HD), lambda h: (0, 2 * H + h, 0, 0)),
            pl.BlockSpec((ng, 1, 1, HD), lambda h: (0, H + h, 0, 0)),
            pl.BlockSpec((ng, 1, 1, HD), lambda h: (0, 2 * H + h, 0, 0)),
            pl.BlockSpec((1, ng), lambda h: (0, 0)),
            pl.BlockSpec((HD, d), lambda h: (h, 0)),
            pl.BlockSpec((1, d), lambda h: (0, 0)),
        ],
        out_specs=pl.BlockSpec((t, d), lambda h: (0, 0)),
        out_shape=jax.ShapeDtypeStruct((t, d), jnp.float32),
        interpret=_INTERPRET,
    )(qkv4, qkv4, qkv4, kvsrc4, kvsrc4, gvneg, wo, bo)


def kernel(x, global_mask, Wq, bq, Wk, bk, Wv, bv, Wo, bo):
    b, t, d = x.shape
    x2 = x[0].astype(jnp.bfloat16)
    wqkv = jnp.concatenate([Wq, Wk, Wv], axis=1).astype(jnp.bfloat16)
    bqkv = jnp.concatenate([bq, bk, bv])[None, :]

    mask = global_mask[0]
    csum = jnp.cumsum(mask.astype(jnp.int32))
    g = csum[-1]
    slots = jnp.where(mask, csum - 1, GMAX + t)
    gidx = (
        jnp.zeros((GMAX,), jnp.int32)
        .at[slots]
        .set(jnp.arange(t, dtype=jnp.int32), mode="drop")
    )
    gvalid_fast = (jnp.arange(GMAX) < g).astype(jnp.float32)[None, :]
    gvalid_slow = mask.astype(jnp.float32)[None, :]

    qkv, kv_glob = _qkv_matmul(x2, wqkv, bqkv, gidx[None, :])

    wo_b = Wo.astype(jnp.bfloat16)
    bo_b = bo[None, :]

    def fast(qkv_):
        return _attention(qkv_, kv_glob, gvalid_fast, wo_b, bo_b)

    def slow(qkv_):
        return _attention(qkv_, qkv_, gvalid_slow, wo_b, bo_b)

    out2 = jax.lax.cond(g <= GMAX, fast, slow, qkv)
    return out2[None]


# speedup vs baseline: 2.3959x; 2.1180x over previous
import functools

import jax
import jax.numpy as jnp
import numpy as np
from jax.experimental import pallas as pl
from jax.experimental.pallas import tpu as pltpu
from jax.experimental.pallas import tpu_sc as plsc

H = 16
HD = 64
WINDOW = 8
NEG = -1e30
TR = 128
LW = 256
GMAX = 128

_INTERPRET = False


def _qkv_kernel(x_ref, w_ref, b_ref, gidx_ref, o_ref, og_ref):
    acc = (
        jnp.dot(x_ref[...], w_ref[...], preferred_element_type=jnp.float32)
        + b_ref[...]
    )
    out = acc.astype(o_ref.dtype)
    o_ref[...] = out
    m = x_ref.shape[0]
    g = gidx_ref.shape[1]
    gcol = jnp.transpose(gidx_ref[...])
    p = (jax.lax.broadcasted_iota(jnp.int32, (g, m), 1) == gcol)
    og_ref[...] = jnp.dot(
        p.astype(jnp.bfloat16), out, preferred_element_type=jnp.float32
    ).astype(og_ref.dtype)


def _qkv_matmul(x, w, b, gidx2, bn=1024):
    m, k = x.shape
    k2, n = w.shape
    g = gidx2.shape[1]
    grid = (n // bn,)
    return pl.pallas_call(
        _qkv_kernel,
        grid=grid,
        in_specs=[
            pl.BlockSpec((m, k), lambda j: (0, 0)),
            pl.BlockSpec((k, bn), lambda j: (0, j)),
            pl.BlockSpec((1, bn), lambda j: (0, j)),
            pl.BlockSpec((1, g), lambda j: (0, 0)),
        ],
        out_specs=[
            pl.BlockSpec((m, bn), lambda j: (0, j)),
            pl.BlockSpec((g, bn), lambda j: (0, j)),
        ],
        out_shape=[
            jax.ShapeDtypeStruct((m, n), jnp.bfloat16),
            jax.ShapeDtypeStruct((g, n), jnp.bfloat16),
        ],
        interpret=_INTERPRET,
    )(x, w, b, gidx2)


def _matmul_kernel(x_ref, w_ref, b_ref, o_ref):
    acc = (
        jnp.dot(x_ref[...], w_ref[...], preferred_element_type=jnp.float32)
        + b_ref[...]
    )
    o_ref[...] = acc.astype(o_ref.dtype)


def _matmul(x, w, b, out_dtype=jnp.float32, bn=1024):
    m, k = x.shape
    k2, n = w.shape
    grid = (n // bn,)
    return pl.pallas_call(
        _matmul_kernel,
        grid=grid,
        in_specs=[
            pl.BlockSpec((m, k), lambda j: (0, 0)),
            pl.BlockSpec((k, bn), lambda j: (0, j)),
            pl.BlockSpec((1, bn), lambda j: (0, j)),
        ],
        out_specs=pl.BlockSpec((m, bn), lambda j: (0, j)),
        out_shape=jax.ShapeDtypeStruct((m, n), out_dtype),
        interpret=_INTERPRET,
    )(x, w, b)


def _band_consts(t):
    out = np.full((3, TR, LW), NEG, np.float32)
    for s, off in enumerate((0, (LW - TR) // 2, LW - TR)):
        i = np.arange(TR)[:, None]
        j = np.arange(LW)[None, :]
        out[s] = np.where(np.abs(j - i - off) <= WINDOW, 0.0, NEG)
    return jnp.asarray(out)


def _attn_kernel(q_ref, k_ref, v_ref, kg_ref, vg_ref, gvneg_ref,
                 wo_ref, bo_ref, o_ref, *, t):
    j = pl.program_id(0)
    nr = t // TR
    ng = kg_ref.shape[0]

    scale = 1.0 / np.sqrt(HD)
    gv = gvneg_ref[...] > -1.0

    ctx_pair = []
    for hh in range(2):
        q = q_ref[:, hh * HD:(hh + 1) * HD]
        k = k_ref[:, hh * HD:(hh + 1) * HD]
        v = v_ref[:, hh * HD:(hh + 1) * HD]
        kg = kg_ref[:, hh * HD:(hh + 1) * HD]
        vg = vg_ref[:, hh * HD:(hh + 1) * HD]

        ctx_rows = []
        for r in range(nr):
            t0 = r * TR
            ls = min(max(t0 - (LW - TR) // 2, 0), t - LW)
            qs = q[t0:t0 + TR]
            kl = k[ls:ls + LW]
            vl = v[ls:ls + LW]

            row_ids = t0 + jax.lax.broadcasted_iota(jnp.int32, (TR, LW), 0)
            key_ids = ls + jax.lax.broadcasted_iota(jnp.int32, (TR, LW), 1)
            band = jnp.abs(key_ids - row_ids) <= WINDOW

            s_loc = jax.lax.dot_general(
                qs, kl, (((1,), (1,)), ((), ())),
                preferred_element_type=jnp.float32,
            ) * scale
            s_loc = jnp.where(band, s_loc, NEG)

            s_g = jax.lax.dot_general(
                qs, kg, (((1,), (1,)), ((), ())),
                preferred_element_type=jnp.float32,
            ) * scale
            s_g = jnp.where(gv, s_g, NEG)

            m = jnp.maximum(
                jnp.max(s_loc, axis=1, keepdims=True),
                jnp.max(s_g, axis=1, keepdims=True),
            )
            p_loc = jnp.exp(s_loc - m)
            p_g = jnp.exp(s_g - m)
            l = (jnp.sum(p_loc, axis=1, keepdims=True)
                 + jnp.sum(p_g, axis=1, keepdims=True))
            acc = (
                jnp.dot(p_loc.astype(jnp.bfloat16), vl,
                        preferred_element_type=jnp.float32)
                + jnp.dot(p_g.astype(jnp.bfloat16), vg,
                          preferred_element_type=jnp.float32)
            )
            ctx_rows.append(acc / l)

        ctx_pair.append(jnp.concatenate(ctx_rows, axis=0))

    ctx = jnp.concatenate(ctx_pair, axis=1)
    contrib = jnp.dot(ctx.astype(jnp.bfloat16), wo_ref[...],
                      preferred_element_type=jnp.float32)

    @pl.when(j == 0)
    def _init():
        o_ref[...] = contrib + bo_ref[...]

    @pl.when(j != 0)
    def _accum():
        o_ref[...] += contrib


def _attention(qkv, kvsrc, gvalid, wo, bo):
    t = qkv.shape[0]
    d = H * HD
    hp = 2 * HD
    ng = gvalid.shape[1]
    gvneg = jnp.where(gvalid > 0.0, 0.0, NEG).astype(jnp.float32)
    nj = H // 2
    grid = (nj,)
    return pl.pallas_call(
        functools.partial(_attn_kernel, t=t),
        grid=grid,
        in_specs=[
            pl.BlockSpec((t, hp), lambda j: (0, j)),
            pl.BlockSpec((t, hp), lambda j: (0, nj + j)),
            pl.BlockSpec((t, hp), lambda j: (0, 2 * nj + j)),
            pl.BlockSpec((ng, hp), lambda j: (0, nj + j)),
            pl.BlockSpec((ng, hp), lambda j: (0, 2 * nj + j)),
            pl.BlockSpec((1, ng), lambda j: (0, 0)),
            pl.BlockSpec((hp, d), lambda j: (j, 0)),
            pl.BlockSpec((1, d), lambda j: (0, 0)),
        ],
        out_specs=pl.BlockSpec((t, d), lambda j: (0, 0)),
        out_shape=jax.ShapeDtypeStruct((t, d), jnp.float32),
        interpret=_INTERPRET,
    )(qkv, qkv, qkv, kvsrc, kvsrc, gvneg, wo, bo)


def kernel(x, global_mask, Wq, bq, Wk, bk, Wv, bv, Wo, bo):
    b, t, d = x.shape
    x2 = x[0].astype(jnp.bfloat16)
    wqkv = jnp.concatenate([Wq, Wk, Wv], axis=1).astype(jnp.bfloat16)
    bqkv = jnp.concatenate([bq, bk, bv])[None, :]

    mask = global_mask[0]
    csum = jnp.cumsum(mask.astype(jnp.int32))
    g = csum[-1]
    slots = jnp.where(mask, csum - 1, GMAX + t)
    gidx = (
        jnp.zeros((GMAX,), jnp.int32)
        .at[slots]
        .set(jnp.arange(t, dtype=jnp.int32), mode="drop")
    )
    gvalid_fast = (jnp.arange(GMAX) < g).astype(jnp.float32)[None, :]
    gvalid_slow = mask.astype(jnp.float32)[None, :]

    qkv, kv_glob = _qkv_matmul(x2, wqkv, bqkv, gidx[None, :])

    wo_b = Wo.astype(jnp.bfloat16)
    bo_b = bo[None, :]

    def fast(qkv_):
        return _attention(qkv_, kv_glob, gvalid_fast, wo_b, bo_b)

    def slow(qkv_):
        return _attention(qkv_, qkv_, gvalid_slow, wo_b, bo_b)

    out2 = jax.lax.cond(g <= GMAX, fast, slow, qkv)
    return out2[None]


# speedup vs baseline: 2.5187x; 1.0513x over previous
import functools

import jax
import jax.numpy as jnp
import numpy as np
from jax.experimental import pallas as pl
from jax.experimental.pallas import tpu as pltpu
from jax.experimental.pallas import tpu_sc as plsc

H = 16
HD = 64
WINDOW = 8
NEG = -1e30
TR = 128
LW = 256
GMAX = 128

_INTERPRET = False


def _qkv_kernel(x_ref, w0_ref, w1_ref, w2_ref, b_ref, gidx_ref, o_ref, og_ref):
    j = pl.program_id(0)
    x = x_ref[...].astype(jnp.bfloat16)
    m = x_ref.shape[0]
    g = gidx_ref.shape[1]
    gcol = jnp.transpose(gidx_ref[...])
    p = (jax.lax.broadcasted_iota(jnp.int32, (g, m), 1) == gcol)
    pb = p.astype(jnp.bfloat16)

    def do(w_ref):
        w = w_ref[...].astype(jnp.bfloat16)
        acc = (
            jnp.dot(x, w, preferred_element_type=jnp.float32) + b_ref[...]
        )
        out = acc.astype(o_ref.dtype)
        o_ref[...] = out
        og_ref[...] = jnp.dot(
            pb, out, preferred_element_type=jnp.float32
        ).astype(og_ref.dtype)

    @pl.when(j == 0)
    def _():
        do(w0_ref)

    @pl.when(j == 1)
    def _():
        do(w1_ref)

    @pl.when(j == 2)
    def _():
        do(w2_ref)


def _qkv_matmul(x, w0, w1, w2, b, gidx2, bn=1024):
    m, k = x.shape
    n = 3 * bn
    g = gidx2.shape[1]
    grid = (3,)
    return pl.pallas_call(
        _qkv_kernel,
        grid=grid,
        in_specs=[
            pl.BlockSpec((m, k), lambda j: (0, 0)),
            pl.BlockSpec((k, bn), lambda j: (0, 0)),
            pl.BlockSpec((k, bn), lambda j: (0, 0)),
            pl.BlockSpec((k, bn), lambda j: (0, 0)),
            pl.BlockSpec((1, bn), lambda j: (0, j)),
            pl.BlockSpec((1, g), lambda j: (0, 0)),
        ],
        out_specs=[
            pl.BlockSpec((m, bn), lambda j: (0, j)),
            pl.BlockSpec((g, bn), lambda j: (0, j)),
        ],
        out_shape=[
            jax.ShapeDtypeStruct((m, n), jnp.bfloat16),
            jax.ShapeDtypeStruct((g, n), jnp.bfloat16),
        ],
        interpret=_INTERPRET,
    )(x, w0, w1, w2, b, gidx2)


def _matmul_kernel(x_ref, w_ref, b_ref, o_ref):
    acc = (
        jnp.dot(x_ref[...], w_ref[...], preferred_element_type=jnp.float32)
        + b_ref[...]
    )
    o_ref[...] = acc.astype(o_ref.dtype)


def _matmul(x, w, b, out_dtype=jnp.float32, bn=1024):
    m, k = x.shape
    k2, n = w.shape
    grid = (n // bn,)
    return pl.pallas_call(
        _matmul_kernel,
        grid=grid,
        in_specs=[
            pl.BlockSpec((m, k), lambda j: (0, 0)),
            pl.BlockSpec((k, bn), lambda j: (0, j)),
            pl.BlockSpec((1, bn), lambda j: (0, j)),
        ],
        out_specs=pl.BlockSpec((m, bn), lambda j: (0, j)),
        out_shape=jax.ShapeDtypeStruct((m, n), out_dtype),
        interpret=_INTERPRET,
    )(x, w, b)


def _band_consts(t):
    out = np.full((3, TR, LW), NEG, np.float32)
    for s, off in enumerate((0, (LW - TR) // 2, LW - TR)):
        i = np.arange(TR)[:, None]
        j = np.arange(LW)[None, :]
        out[s] = np.where(np.abs(j - i - off) <= WINDOW, 0.0, NEG)
    return jnp.asarray(out)


def _attn_kernel(q_ref, k_ref, v_ref, kg_ref, vg_ref, gvneg_ref,
                 wo_ref, bo_ref, o_ref, *, t):
    j = pl.program_id(0)
    nr = t // TR
    ng = kg_ref.shape[0]

    scale = 1.0 / np.sqrt(HD)
    gv = gvneg_ref[...] > -1.0

    ctx_pair = []
    for hh in range(2):
        q = q_ref[:, hh * HD:(hh + 1) * HD]
        k = k_ref[:, hh * HD:(hh + 1) * HD]
        v = v_ref[:, hh * HD:(hh + 1) * HD]
        kg = kg_ref[:, hh * HD:(hh + 1) * HD]
        vg = vg_ref[:, hh * HD:(hh + 1) * HD]

        ctx_rows = []
        for r in range(nr):
            t0 = r * TR
            ls = min(max(t0 - (LW - TR) // 2, 0), t - LW)
            qs = q[t0:t0 + TR]
            kl = k[ls:ls + LW]
            vl = v[ls:ls + LW]

            row_ids = t0 + jax.lax.broadcasted_iota(jnp.int32, (TR, LW), 0)
            key_ids = ls + jax.lax.broadcasted_iota(jnp.int32, (TR, LW), 1)
            band = jnp.abs(key_ids - row_ids) <= WINDOW

            s_loc = jax.lax.dot_general(
                qs, kl, (((1,), (1,)), ((), ())),
                preferred_element_type=jnp.float32,
            ) * scale
            s_loc = jnp.where(band, s_loc, NEG)

            s_g = jax.lax.dot_general(
                qs, kg, (((1,), (1,)), ((), ())),
                preferred_element_type=jnp.float32,
            ) * scale
            s_g = jnp.where(gv, s_g, NEG)

            m = jnp.maximum(
                jnp.max(s_loc, axis=1, keepdims=True),
                jnp.max(s_g, axis=1, keepdims=True),
            )
            p_loc = jnp.exp(s_loc - m)
            p_g = jnp.exp(s_g - m)
            l = (jnp.sum(p_loc, axis=1, keepdims=True)
                 + jnp.sum(p_g, axis=1, keepdims=True))
            acc = (
                jnp.dot(p_loc.astype(jnp.bfloat16), vl,
                        preferred_element_type=jnp.float32)
                + jnp.dot(p_g.astype(jnp.bfloat16), vg,
                          preferred_element_type=jnp.float32)
            )
            ctx_rows.append(acc / l)

        ctx_pair.append(jnp.concatenate(ctx_rows, axis=0))

    ctx = jnp.concatenate(ctx_pair, axis=1)
    contrib = jnp.dot(ctx.astype(jnp.bfloat16),
                      wo_ref[...].astype(jnp.bfloat16),
                      preferred_element_type=jnp.float32)

    @pl.when(j == 0)
    def _init():
        o_ref[...] = contrib + bo_ref[...]

    @pl.when(j != 0)
    def _accum():
        o_ref[...] += contrib


def _attention(qkv, kvsrc, gvalid, wo, bo):
    t = qkv.shape[0]
    d = H * HD
    hp = 2 * HD
    ng = gvalid.shape[1]
    gvneg = jnp.where(gvalid > 0.0, 0.0, NEG).astype(jnp.float32)
    nj = H // 2
    grid = (nj,)
    return pl.pallas_call(
        functools.partial(_attn_kernel, t=t),
        grid=grid,
        in_specs=[
            pl.BlockSpec((t, hp), lambda j: (0, j)),
            pl.BlockSpec((t, hp), lambda j: (0, nj + j)),
            pl.BlockSpec((t, hp), lambda j: (0, 2 * nj + j)),
            pl.BlockSpec((ng, hp), lambda j: (0, nj + j)),
            pl.BlockSpec((ng, hp), lambda j: (0, 2 * nj + j)),
            pl.BlockSpec((1, ng), lambda j: (0, 0)),
            pl.BlockSpec((hp, d), lambda j: (j, 0)),
            pl.BlockSpec((1, d), lambda j: (0, 0)),
        ],
        out_specs=pl.BlockSpec((t, d), lambda j: (0, 0)),
        out_shape=jax.ShapeDtypeStruct((t, d), jnp.float32),
        interpret=_INTERPRET,
    )(qkv, qkv, qkv, kvsrc, kvsrc, gvneg, wo, bo)


def kernel(x, global_mask, Wq, bq, Wk, bk, Wv, bv, Wo, bo):
    b, t, d = x.shape
    x2 = x[0]
    bqkv = jnp.concatenate([bq, bk, bv])[None, :]

    mask = global_mask[0]
    csum = jnp.cumsum(mask.astype(jnp.int32))
    g = csum[-1]
    slots = jnp.where(mask, csum - 1, GMAX + t)
    gidx = (
        jnp.zeros((GMAX,), jnp.int32)
        .at[slots]
        .set(jnp.arange(t, dtype=jnp.int32), mode="drop")
    )
    gvalid_fast = (jnp.arange(GMAX) < g).astype(jnp.float32)[None, :]
    gvalid_slow = mask.astype(jnp.float32)[None, :]

    qkv, kv_glob = _qkv_matmul(x2, Wq, Wk, Wv, bqkv, gidx[None, :])

    wo_b = Wo
    bo_b = bo[None, :]

    def fast(qkv_):
        return _attention(qkv_, kv_glob, gvalid_fast, wo_b, bo_b)

    def slow(qkv_):
        return _attention(qkv_, qkv_, gvalid_slow, wo_b, bo_b)

    out2 = jax.lax.cond(g <= GMAX, fast, slow, qkv)
    return out2[None]


# speedup vs baseline: 2.7484x; 1.0912x over previous
import functools

import jax
import jax.numpy as jnp
import numpy as np
from jax.experimental import pallas as pl
from jax.experimental.pallas import tpu as pltpu
from jax.experimental.pallas import tpu_sc as plsc

H = 16
HD = 64
WINDOW = 8
NEG = -1e30
TR = 128
LW = 256
GMAX = 128

_INTERPRET = False


def _qkv_kernel(x_ref, w0_ref, w1_ref, w2_ref, b_ref, mask_ref,
                o_ref, og_ref, gvneg_ref, gcnt_ref, gcol_v):
    j = pl.program_id(0)
    m = x_ref.shape[0]
    g = og_ref.shape[0]

    @pl.when(j == 0)
    def _compact():
        mk = mask_ref[...]
        csum = mk
        sh = 1
        while sh < m:
            shifted = jnp.concatenate(
                [jnp.zeros((1, sh), jnp.float32), csum[:, :m - sh]], axis=1
            )
            csum = csum + shifted
            sh *= 2
        cnt = csum[0, m - 1].astype(jnp.int32)
        gcnt_ref[0, 0] = cnt
        csum_i = csum.astype(jnp.int32)
        s_col = jax.lax.broadcasted_iota(jnp.int32, (g, m), 0) + 1
        e = jnp.where(
            (csum_i == s_col) & (mk > 0.0),
            jax.lax.broadcasted_iota(jnp.int32, (g, m), 1),
            0,
        )
        gcol_v[...] = jnp.sum(e, axis=1, keepdims=True)
        s_row = jax.lax.broadcasted_iota(jnp.int32, (1, g), 1)
        gvneg_ref[...] = jnp.where(s_row < cnt, 0.0, NEG)

    pb = (
        jax.lax.broadcasted_iota(jnp.int32, (g, m), 1) == gcol_v[...]
    ).astype(jnp.bfloat16)
    x = x_ref[...].astype(jnp.bfloat16)

    def do(w_ref):
        w = w_ref[...].astype(jnp.bfloat16)
        acc = (
            jnp.dot(x, w, preferred_element_type=jnp.float32) + b_ref[...]
        )
        out = acc.astype(o_ref.dtype)
        o_ref[...] = out
        og_ref[...] = jnp.dot(
            pb, out, preferred_element_type=jnp.float32
        ).astype(og_ref.dtype)

    @pl.when(j == 0)
    def _():
        do(w0_ref)

    @pl.when(j == 1)
    def _():
        do(w1_ref)

    @pl.when(j == 2)
    def _():
        do(w2_ref)


def _qkv_matmul(x, w0, w1, w2, b, maskf, bn=1024):
    m, k = x.shape
    n = 3 * bn
    g = GMAX
    grid = (3,)
    return pl.pallas_call(
        _qkv_kernel,
        grid=grid,
        in_specs=[
            pl.BlockSpec((m, k), lambda j: (0, 0)),
            pl.BlockSpec((k, bn), lambda j: (0, 0)),
            pl.BlockSpec((k, bn), lambda j: (0, 0)),
            pl.BlockSpec((k, bn), lambda j: (0, 0)),
            pl.BlockSpec((1, bn), lambda j: (0, j)),
            pl.BlockSpec((1, m), lambda j: (0, 0)),
        ],
        out_specs=[
            pl.BlockSpec((m, bn), lambda j: (0, j)),
            pl.BlockSpec((g, bn), lambda j: (0, j)),
            pl.BlockSpec((1, g), lambda j: (0, 0)),
            pl.BlockSpec((1, 1), lambda j: (0, 0), memory_space=pltpu.SMEM),
        ],
        out_shape=[
            jax.ShapeDtypeStruct((m, n), jnp.bfloat16),
            jax.ShapeDtypeStruct((g, n), jnp.bfloat16),
            jax.ShapeDtypeStruct((1, g), jnp.float32),
            jax.ShapeDtypeStruct((1, 1), jnp.int32),
        ],
        scratch_shapes=[pltpu.VMEM((g, 1), jnp.int32)],
        interpret=_INTERPRET,
    )(x, w0, w1, w2, b, maskf)


def _matmul_kernel(x_ref, w_ref, b_ref, o_ref):
    acc = (
        jnp.dot(x_ref[...], w_ref[...], preferred_element_type=jnp.float32)
        + b_ref[...]
    )
    o_ref[...] = acc.astype(o_ref.dtype)


def _matmul(x, w, b, out_dtype=jnp.float32, bn=1024):
    m, k = x.shape
    k2, n = w.shape
    grid = (n // bn,)
    return pl.pallas_call(
        _matmul_kernel,
        grid=grid,
        in_specs=[
            pl.BlockSpec((m, k), lambda j: (0, 0)),
            pl.BlockSpec((k, bn), lambda j: (0, j)),
            pl.BlockSpec((1, bn), lambda j: (0, j)),
        ],
        out_specs=pl.BlockSpec((m, bn), lambda j: (0, j)),
        out_shape=jax.ShapeDtypeStruct((m, n), out_dtype),
        interpret=_INTERPRET,
    )(x, w, b)


def _band_consts(t):
    out = np.full((3, TR, LW), NEG, np.float32)
    for s, off in enumerate((0, (LW - TR) // 2, LW - TR)):
        i = np.arange(TR)[:, None]
        j = np.arange(LW)[None, :]
        out[s] = np.where(np.abs(j - i - off) <= WINDOW, 0.0, NEG)
    return jnp.asarray(out)


def _attn_kernel(q_ref, k_ref, v_ref, kg_ref, vg_ref, gvneg_ref,
                 wo_ref, bo_ref, o_ref, *, t):
    j = pl.program_id(0)
    nr = t // TR
    ng = kg_ref.shape[0]

    scale = 1.0 / np.sqrt(HD)
    gv = gvneg_ref[...] > -1.0

    ctx_pair = []
    for hh in range(2):
        q = q_ref[:, hh * HD:(hh + 1) * HD]
        k = k_ref[:, hh * HD:(hh + 1) * HD]
        v = v_ref[:, hh * HD:(hh + 1) * HD]
        kg = kg_ref[:, hh * HD:(hh + 1) * HD]
        vg = vg_ref[:, hh * HD:(hh + 1) * HD]

        ctx_rows = []
        for r in range(nr):
            t0 = r * TR
            ls = min(max(t0 - (LW - TR) // 2, 0), t - LW)
            qs = q[t0:t0 + TR]
            kl = k[ls:ls + LW]
            vl = v[ls:ls + LW]

            row_ids = t0 + jax.lax.broadcasted_iota(jnp.int32, (TR, LW), 0)
            key_ids = ls + jax.lax.broadcasted_iota(jnp.int32, (TR, LW), 1)
            band = jnp.abs(key_ids - row_ids) <= WINDOW

            s_loc = jax.lax.dot_general(
                qs, kl, (((1,), (1,)), ((), ())),
                preferred_element_type=jnp.float32,
            ) * scale
            s_loc = jnp.where(band, s_loc, NEG)

            s_g = jax.lax.dot_general(
                qs, kg, (((1,), (1,)), ((), ())),
                preferred_element_type=jnp.float32,
            ) * scale
            s_g = jnp.where(gv, s_g, NEG)

            m = jnp.maximum(
                jnp.max(s_loc, axis=1, keepdims=True),
                jnp.max(s_g, axis=1, keepdims=True),
            )
            p_loc = jnp.exp(s_loc - m)
            p_g = jnp.exp(s_g - m)
            l = (jnp.sum(p_loc, axis=1, keepdims=True)
                 + jnp.sum(p_g, axis=1, keepdims=True))
            acc = (
                jnp.dot(p_loc.astype(jnp.bfloat16), vl,
                        preferred_element_type=jnp.float32)
                + jnp.dot(p_g.astype(jnp.bfloat16), vg,
                          preferred_element_type=jnp.float32)
            )
            ctx_rows.append(acc / l)

        ctx_pair.append(jnp.concatenate(ctx_rows, axis=0))

    ctx = jnp.concatenate(ctx_pair, axis=1)
    contrib = jnp.dot(ctx.astype(jnp.bfloat16),
                      wo_ref[...].astype(jnp.bfloat16),
                      preferred_element_type=jnp.float32)

    @pl.when(j == 0)
    def _init():
        o_ref[...] = contrib + bo_ref[...]

    @pl.when(j != 0)
    def _accum():
        o_ref[...] += contrib


def _attention(qkv, kvsrc, gvneg, wo, bo):
    t = qkv.shape[0]
    d = H * HD
    hp = 2 * HD
    ng = gvneg.shape[1]
    nj = H // 2
    grid = (nj,)
    return pl.pallas_call(
        functools.partial(_attn_kernel, t=t),
        grid=grid,
        in_specs=[
            pl.BlockSpec((t, hp), lambda j: (0, j)),
            pl.BlockSpec((t, hp), lambda j: (0, nj + j)),
            pl.BlockSpec((t, hp), lambda j: (0, 2 * nj + j)),
            pl.BlockSpec((ng, hp), lambda j: (0, nj + j)),
            pl.BlockSpec((ng, hp), lambda j: (0, 2 * nj + j)),
            pl.BlockSpec((1, ng), lambda j: (0, 0)),
            pl.BlockSpec((hp, d), lambda j: (j, 0)),
            pl.BlockSpec((1, d), lambda j: (0, 0)),
        ],
        out_specs=pl.BlockSpec((t, d), lambda j: (0, 0)),
        out_shape=jax.ShapeDtypeStruct((t, d), jnp.float32),
        interpret=_INTERPRET,
    )(qkv, qkv, qkv, kvsrc, kvsrc, gvneg, wo, bo)


def kernel(x, global_mask, Wq, bq, Wk, bk, Wv, bv, Wo, bo):
    b, t, d = x.shape
    x2 = x[0]
    bqkv = jnp.concatenate([bq, bk, bv])[None, :]

    maskf = global_mask.astype(jnp.float32)
    qkv, kv_glob, gvneg_fast, gcnt = _qkv_matmul(x2, Wq, Wk, Wv, bqkv, maskf)
    gvneg_slow = jnp.where(maskf > 0.0, 0.0, NEG)

    wo_b = Wo
    bo_b = bo[None, :]

    def fast(qkv_):
        return _attention(qkv_, kv_glob, gvneg_fast, wo_b, bo_b)

    def slow(qkv_):
        return _attention(qkv_, qkv_, gvneg_slow, wo_b, bo_b)

    out2 = jax.lax.cond(gcnt[0, 0] <= GMAX, fast, slow, qkv)
    return out2[None]


# speedup vs baseline: 3.3554x; 1.2209x over previous
import functools

import jax
import jax.numpy as jnp
import numpy as np
from jax.experimental import pallas as pl
from jax.experimental.pallas import tpu as pltpu
from jax.experimental.pallas import tpu_sc as plsc

H = 16
HD = 64
WINDOW = 8
NEG = -1e30
TR = 256
LW = 384
GMAX = 128

_INTERPRET = False


def _qkv_kernel(x_ref, w0_ref, w1_ref, w2_ref, b_ref, mask_ref,
                o_ref, og_ref, gvneg_ref, gcnt_ref, gcol_v):
    j = pl.program_id(0)
    m = x_ref.shape[0]
    g = og_ref.shape[0]

    @pl.when(j == 0)
    def _compact():
        mk = mask_ref[...]
        csum = mk
        sh = 1
        while sh < m:
            shifted = jnp.concatenate(
                [jnp.zeros((1, sh), jnp.float32), csum[:, :m - sh]], axis=1
            )
            csum = csum + shifted
            sh *= 2
        cnt = csum[0, m - 1].astype(jnp.int32)
        gcnt_ref[0, 0] = cnt
        csum_i = csum.astype(jnp.int32)
        s_col = jax.lax.broadcasted_iota(jnp.int32, (g, m), 0) + 1
        e = jnp.where(
            (csum_i == s_col) & (mk > 0.0),
            jax.lax.broadcasted_iota(jnp.int32, (g, m), 1),
            0,
        )
        gcol_v[...] = jnp.sum(e, axis=1, keepdims=True)
        s_row = jax.lax.broadcasted_iota(jnp.int32, (1, g), 1)
        gvneg_ref[...] = jnp.where(s_row < cnt, 0.0, NEG)

    pb = (
        jax.lax.broadcasted_iota(jnp.int32, (g, m), 1) == gcol_v[...]
    ).astype(jnp.bfloat16)
    x = x_ref[...].astype(jnp.bfloat16)

    def do(w_ref):
        w = w_ref[...].astype(jnp.bfloat16)
        acc = (
            jnp.dot(x, w, preferred_element_type=jnp.float32) + b_ref[...]
        )
        out = acc.astype(o_ref.dtype)
        o_ref[...] = out
        og_ref[...] = jnp.dot(
            pb, out, preferred_element_type=jnp.float32
        ).astype(og_ref.dtype)

    @pl.when(j == 0)
    def _():
        do(w0_ref)

    @pl.when(j == 1)
    def _():
        do(w1_ref)

    @pl.when(j == 2)
    def _():
        do(w2_ref)


def _qkv_matmul(x, w0, w1, w2, b, maskf, bn=1024):
    m, k = x.shape
    n = 3 * bn
    g = GMAX
    grid = (3,)
    return pl.pallas_call(
        _qkv_kernel,
        grid=grid,
        in_specs=[
            pl.BlockSpec((m, k), lambda j: (0, 0)),
            pl.BlockSpec((k, bn), lambda j: (0, 0)),
            pl.BlockSpec((k, bn), lambda j: (0, 0)),
            pl.BlockSpec((k, bn), lambda j: (0, 0)),
            pl.BlockSpec((1, bn), lambda j: (0, j)),
            pl.BlockSpec((1, m), lambda j: (0, 0)),
        ],
        out_specs=[
            pl.BlockSpec((m, bn), lambda j: (0, j)),
            pl.BlockSpec((g, bn), lambda j: (0, j)),
            pl.BlockSpec((1, g), lambda j: (0, 0)),
            pl.BlockSpec((1, 1), lambda j: (0, 0), memory_space=pltpu.SMEM),
        ],
        out_shape=[
            jax.ShapeDtypeStruct((m, n), jnp.bfloat16),
            jax.ShapeDtypeStruct((g, n), jnp.bfloat16),
            jax.ShapeDtypeStruct((1, g), jnp.float32),
            jax.ShapeDtypeStruct((1, 1), jnp.int32),
        ],
        scratch_shapes=[pltpu.VMEM((g, 1), jnp.int32)],
        interpret=_INTERPRET,
    )(x, w0, w1, w2, b, maskf)


def _matmul_kernel(x_ref, w_ref, b_ref, o_ref):
    acc = (
        jnp.dot(x_ref[...], w_ref[...], preferred_element_type=jnp.float32)
        + b_ref[...]
    )
    o_ref[...] = acc.astype(o_ref.dtype)


def _matmul(x, w, b, out_dtype=jnp.float32, bn=1024):
    m, k = x.shape
    k2, n = w.shape
    grid = (n // bn,)
    return pl.pallas_call(
        _matmul_kernel,
        grid=grid,
        in_specs=[
            pl.BlockSpec((m, k), lambda j: (0, 0)),
            pl.BlockSpec((k, bn), lambda j: (0, j)),
            pl.BlockSpec((1, bn), lambda j: (0, j)),
        ],
        out_specs=pl.BlockSpec((m, bn), lambda j: (0, j)),
        out_shape=jax.ShapeDtypeStruct((m, n), out_dtype),
        interpret=_INTERPRET,
    )(x, w, b)


def _band_consts(t):
    out = np.full((3, TR, LW), NEG, np.float32)
    for s, off in enumerate((0, (LW - TR) // 2, LW - TR)):
        i = np.arange(TR)[:, None]
        j = np.arange(LW)[None, :]
        out[s] = np.where(np.abs(j - i - off) <= WINDOW, 0.0, NEG)
    return jnp.asarray(out)


def _attn_kernel(q_ref, k_ref, v_ref, kg_ref, vg_ref, gvneg_ref,
                 wo_ref, bo_ref, o_ref, *, t):
    j = pl.program_id(0)
    nr = t // TR
    ng = kg_ref.shape[0]

    scale = 1.0 / np.sqrt(HD)
    gv = gvneg_ref[...] > -1.0

    ctx_pair = []
    for hh in range(2):
        q = q_ref[:, hh * HD:(hh + 1) * HD]
        k = k_ref[:, hh * HD:(hh + 1) * HD]
        v = v_ref[:, hh * HD:(hh + 1) * HD]
        kg = kg_ref[:, hh * HD:(hh + 1) * HD]
        vg = vg_ref[:, hh * HD:(hh + 1) * HD]

        ctx_rows = []
        for r in range(nr):
            t0 = r * TR
            ls = min(max(t0 - (LW - TR) // 2, 0), t - LW)
            qs = q[t0:t0 + TR]
            kl = k[ls:ls + LW]
            vl = v[ls:ls + LW]

            row_ids = t0 + jax.lax.broadcasted_iota(jnp.int32, (TR, LW), 0)
            key_ids = ls + jax.lax.broadcasted_iota(jnp.int32, (TR, LW), 1)
            band = jnp.abs(key_ids - row_ids) <= WINDOW

            s_loc = jax.lax.dot_general(
                qs, kl, (((1,), (1,)), ((), ())),
                preferred_element_type=jnp.float32,
            ) * scale
            s_loc = jnp.where(band, s_loc, NEG)

            s_g = jax.lax.dot_general(
                qs, kg, (((1,), (1,)), ((), ())),
                preferred_element_type=jnp.float32,
            ) * scale
            s_g = jnp.where(gv, s_g, NEG)

            m = jnp.maximum(
                jnp.max(s_loc, axis=1, keepdims=True),
                jnp.max(s_g, axis=1, keepdims=True),
            )
            p_loc = jnp.exp(s_loc - m)
            p_g = jnp.exp(s_g - m)
            l = (jnp.sum(p_loc, axis=1, keepdims=True)
                 + jnp.sum(p_g, axis=1, keepdims=True))
            acc = (
                jnp.dot(p_loc.astype(jnp.bfloat16), vl,
                        preferred_element_type=jnp.float32)
                + jnp.dot(p_g.astype(jnp.bfloat16), vg,
                          preferred_element_type=jnp.float32)
            )
            ctx_rows.append(acc / l)

        ctx_pair.append(jnp.concatenate(ctx_rows, axis=0))

    ctx = jnp.concatenate(ctx_pair, axis=1)
    contrib = jnp.dot(ctx.astype(jnp.bfloat16),
                      wo_ref[...].astype(jnp.bfloat16),
                      preferred_element_type=jnp.float32)

    @pl.when(j == 0)
    def _init():
        o_ref[...] = contrib + bo_ref[...]

    @pl.when(j != 0)
    def _accum():
        o_ref[...] += contrib


def _attention(qkv, kvsrc, gvneg, wo, bo):
    t = qkv.shape[0]
    d = H * HD
    hp = 2 * HD
    ng = gvneg.shape[1]
    nj = H // 2
    grid = (nj,)
    return pl.pallas_call(
        functools.partial(_attn_kernel, t=t),
        grid=grid,
        in_specs=[
            pl.BlockSpec((t, hp), lambda j: (0, j)),
            pl.BlockSpec((t, hp), lambda j: (0, nj + j)),
            pl.BlockSpec((t, hp), lambda j: (0, 2 * nj + j)),
            pl.BlockSpec((ng, hp), lambda j: (0, nj + j)),
            pl.BlockSpec((ng, hp), lambda j: (0, 2 * nj + j)),
            pl.BlockSpec((1, ng), lambda j: (0, 0)),
            pl.BlockSpec((hp, d), lambda j: (j, 0)),
            pl.BlockSpec((1, d), lambda j: (0, 0)),
        ],
        out_specs=pl.BlockSpec((t, d), lambda j: (0, 0)),
        out_shape=jax.ShapeDtypeStruct((t, d), jnp.float32),
        interpret=_INTERPRET,
    )(qkv, qkv, qkv, kvsrc, kvsrc, gvneg, wo, bo)


def kernel(x, global_mask, Wq, bq, Wk, bk, Wv, bv, Wo, bo):
    b, t, d = x.shape
    x2 = x[0]
    bqkv = jnp.concatenate([bq, bk, bv])[None, :]

    maskf = global_mask.astype(jnp.float32)
    qkv, kv_glob, gvneg_fast, gcnt = _qkv_matmul(x2, Wq, Wk, Wv, bqkv, maskf)
    gvneg_slow = jnp.where(maskf > 0.0, 0.0, NEG)

    wo_b = Wo
    bo_b = bo[None, :]

    def fast(qkv_):
        return _attention(qkv_, kv_glob, gvneg_fast, wo_b, bo_b)

    def slow(qkv_):
        return _attention(qkv_, qkv_, gvneg_slow, wo_b, bo_b)

    out2 = jax.lax.cond(gcnt[0, 0] <= GMAX, fast, slow, qkv)
    return out2[None]


# speedup vs baseline: 3.5196x; 1.0489x over previous
import functools

import jax
import jax.numpy as jnp
import numpy as np
from jax.experimental import pallas as pl
from jax.experimental.pallas import tpu as pltpu
from jax.experimental.pallas import tpu_sc as plsc

H = 16
HD = 64
WINDOW = 8
NEG = -1e30
TR = 512
LW = 640
GMAX = 128

_INTERPRET = False


def _qkv_kernel(x_ref, w0_ref, w1_ref, w2_ref, b_ref, mask_ref,
                o_ref, og_ref, gvneg_ref, gcnt_ref, gcol_v):
    j = pl.program_id(0)
    m = x_ref.shape[0]
    g = og_ref.shape[0]

    @pl.when(j == 0)
    def _compact():
        mk = mask_ref[...]
        csum = mk
        sh = 1
        while sh < m:
            shifted = jnp.concatenate(
                [jnp.zeros((1, sh), jnp.float32), csum[:, :m - sh]], axis=1
            )
            csum = csum + shifted
            sh *= 2
        cnt = csum[0, m - 1].astype(jnp.int32)
        gcnt_ref[0, 0] = cnt
        csum_i = csum.astype(jnp.int32)
        s_col = jax.lax.broadcasted_iota(jnp.int32, (g, m), 0) + 1
        e = jnp.where(
            (csum_i == s_col) & (mk > 0.0),
            jax.lax.broadcasted_iota(jnp.int32, (g, m), 1),
            0,
        )
        gcol_v[...] = jnp.sum(e, axis=1, keepdims=True)
        s_row = jax.lax.broadcasted_iota(jnp.int32, (1, g), 1)
        gvneg_ref[...] = jnp.where(s_row < cnt, 0.0, NEG)

    pb = (
        jax.lax.broadcasted_iota(jnp.int32, (g, m), 1) == gcol_v[...]
    ).astype(jnp.bfloat16)
    x = x_ref[...].astype(jnp.bfloat16)

    def do(w_ref):
        w = w_ref[...].astype(jnp.bfloat16)
        acc = (
            jnp.dot(x, w, preferred_element_type=jnp.float32) + b_ref[...]
        )
        out = acc.astype(o_ref.dtype)
        o_ref[...] = out
        og_ref[...] = jnp.dot(
            pb, out, preferred_element_type=jnp.float32
        ).astype(og_ref.dtype)

    @pl.when(j == 0)
    def _():
        do(w0_ref)

    @pl.when(j == 1)
    def _():
        do(w1_ref)

    @pl.when(j == 2)
    def _():
        do(w2_ref)


def _qkv_matmul(x, w0, w1, w2, b, maskf, bn=1024):
    m, k = x.shape
    n = 3 * bn
    g = GMAX
    grid = (3,)
    return pl.pallas_call(
        _qkv_kernel,
        grid=grid,
        in_specs=[
            pl.BlockSpec((m, k), lambda j: (0, 0)),
            pl.BlockSpec((k, bn), lambda j: (0, 0)),
            pl.BlockSpec((k, bn), lambda j: (0, 0)),
            pl.BlockSpec((k, bn), lambda j: (0, 0)),
            pl.BlockSpec((1, bn), lambda j: (0, j)),
            pl.BlockSpec((1, m), lambda j: (0, 0)),
        ],
        out_specs=[
            pl.BlockSpec((m, bn), lambda j: (0, j)),
            pl.BlockSpec((g, bn), lambda j: (0, j)),
            pl.BlockSpec((1, g), lambda j: (0, 0)),
            pl.BlockSpec((1, 1), lambda j: (0, 0), memory_space=pltpu.SMEM),
        ],
        out_shape=[
            jax.ShapeDtypeStruct((m, n), jnp.bfloat16),
            jax.ShapeDtypeStruct((g, n), jnp.bfloat16),
            jax.ShapeDtypeStruct((1, g), jnp.float32),
            jax.ShapeDtypeStruct((1, 1), jnp.int32),
        ],
        scratch_shapes=[pltpu.VMEM((g, 1), jnp.int32)],
        interpret=_INTERPRET,
    )(x, w0, w1, w2, b, maskf)


def _matmul_kernel(x_ref, w_ref, b_ref, o_ref):
    acc = (
        jnp.dot(x_ref[...], w_ref[...], preferred_element_type=jnp.float32)
        + b_ref[...]
    )
    o_ref[...] = acc.astype(o_ref.dtype)


def _matmul(x, w, b, out_dtype=jnp.float32, bn=1024):
    m, k = x.shape
    k2, n = w.shape
    grid = (n // bn,)
    return pl.pallas_call(
        _matmul_kernel,
        grid=grid,
        in_specs=[
            pl.BlockSpec((m, k), lambda j: (0, 0)),
            pl.BlockSpec((k, bn), lambda j: (0, j)),
            pl.BlockSpec((1, bn), lambda j: (0, j)),
        ],
        out_specs=pl.BlockSpec((m, bn), lambda j: (0, j)),
        out_shape=jax.ShapeDtypeStruct((m, n), out_dtype),
        interpret=_INTERPRET,
    )(x, w, b)


def _band_consts(t):
    out = np.full((3, TR, LW), NEG, np.float32)
    for s, off in enumerate((0, (LW - TR) // 2, LW - TR)):
        i = np.arange(TR)[:, None]
        j = np.arange(LW)[None, :]
        out[s] = np.where(np.abs(j - i - off) <= WINDOW, 0.0, NEG)
    return jnp.asarray(out)


def _attn_kernel(q_ref, k_ref, v_ref, kg_ref, vg_ref, gvneg_ref,
                 wo_ref, bo_ref, o_ref, *, t):
    j = pl.program_id(0)
    nr = t // TR
    ng = kg_ref.shape[0]

    scale = 1.0 / np.sqrt(HD)
    gv = gvneg_ref[...] > -1.0

    ctx_pair = []
    for hh in range(2):
        q = q_ref[:, hh * HD:(hh + 1) * HD]
        k = k_ref[:, hh * HD:(hh + 1) * HD]
        v = v_ref[:, hh * HD:(hh + 1) * HD]
        kg = kg_ref[:, hh * HD:(hh + 1) * HD]
        vg = vg_ref[:, hh * HD:(hh + 1) * HD]

        ctx_rows = []
        for r in range(nr):
            t0 = r * TR
            ls = min(max(t0 - (LW - TR) // 2, 0), t - LW)
            qs = q[t0:t0 + TR]
            kl = k[ls:ls + LW]
            vl = v[ls:ls + LW]

            row_ids = t0 + jax.lax.broadcasted_iota(jnp.int32, (TR, LW), 0)
            key_ids = ls + jax.lax.broadcasted_iota(jnp.int32, (TR, LW), 1)
            band = jnp.abs(key_ids - row_ids) <= WINDOW

            s_loc = jax.lax.dot_general(
                qs, kl, (((1,), (1,)), ((), ())),
                preferred_element_type=jnp.float32,
            ) * scale
            s_loc = jnp.where(band, s_loc, NEG)

            s_g = jax.lax.dot_general(
                qs, kg, (((1,), (1,)), ((), ())),
                preferred_element_type=jnp.float32,
            ) * scale
            s_g = jnp.where(gv, s_g, NEG)

            m = jnp.maximum(
                jnp.max(s_loc, axis=1, keepdims=True),
                jnp.max(s_g, axis=1, keepdims=True),
            )
            p_loc = jnp.exp(s_loc - m)
            p_g = jnp.exp(s_g - m)
            l = (jnp.sum(p_loc, axis=1, keepdims=True)
                 + jnp.sum(p_g, axis=1, keepdims=True))
            acc = (
                jnp.dot(p_loc.astype(jnp.bfloat16), vl,
                        preferred_element_type=jnp.float32)
                + jnp.dot(p_g.astype(jnp.bfloat16), vg,
                          preferred_element_type=jnp.float32)
            )
            ctx_rows.append(acc / l)

        ctx_pair.append(jnp.concatenate(ctx_rows, axis=0))

    ctx = jnp.concatenate(ctx_pair, axis=1)
    contrib = jnp.dot(ctx.astype(jnp.bfloat16),
                      wo_ref[...].astype(jnp.bfloat16),
                      preferred_element_type=jnp.float32)

    @pl.when(j == 0)
    def _init():
        o_ref[...] = contrib + bo_ref[...]

    @pl.when(j != 0)
    def _accum():
        o_ref[...] += contrib


def _attention(qkv, kvsrc, gvneg, wo, bo):
    t = qkv.shape[0]
    d = H * HD
    hp = 2 * HD
    ng = gvneg.shape[1]
    nj = H // 2
    grid = (nj,)
    return pl.pallas_call(
        functools.partial(_attn_kernel, t=t),
        grid=grid,
        in_specs=[
            pl.BlockSpec((t, hp), lambda j: (0, j)),
            pl.BlockSpec((t, hp), lambda j: (0, nj + j)),
            pl.BlockSpec((t, hp), lambda j: (0, 2 * nj + j)),
            pl.BlockSpec((ng, hp), lambda j: (0, nj + j)),
            pl.BlockSpec((ng, hp), lambda j: (0, 2 * nj + j)),
            pl.BlockSpec((1, ng), lambda j: (0, 0)),
            pl.BlockSpec((hp, d), lambda j: (j, 0)),
            pl.BlockSpec((1, d), lambda j: (0, 0)),
        ],
        out_specs=pl.BlockSpec((t, d), lambda j: (0, 0)),
        out_shape=jax.ShapeDtypeStruct((t, d), jnp.float32),
        interpret=_INTERPRET,
    )(qkv, qkv, qkv, kvsrc, kvsrc, gvneg, wo, bo)


def kernel(x, global_mask, Wq, bq, Wk, bk, Wv, bv, Wo, bo):
    b, t, d = x.shape
    x2 = x[0]
    bqkv = jnp.concatenate([bq, bk, bv])[None, :]

    maskf = global_mask.astype(jnp.float32)
    qkv, kv_glob, gvneg_fast, gcnt = _qkv_matmul(x2, Wq, Wk, Wv, bqkv, maskf)
    gvneg_slow = jnp.where(maskf > 0.0, 0.0, NEG)

    wo_b = Wo
    bo_b = bo[None, :]

    def fast(qkv_):
        return _attention(qkv_, kv_glob, gvneg_fast, wo_b, bo_b)

    def slow(qkv_):
        return _attention(qkv_, qkv_, gvneg_slow, wo_b, bo_b)

    out2 = jax.lax.cond(gcnt[0, 0] <= GMAX, fast, slow, qkv)
    return out2[None]
